# Initial kernel scaffold; baseline (speedup 1.0000x reference)
import jax
import jax.numpy as jnp
from jax.experimental import pallas as pl


def kernel(item_indices, neighbor_indices, item_table, entity_table, fc1_w, fc1_b):
    raise NotImplementedError("write your pallas kernel here")


# speedup vs baseline: 2.2008x; 2.2008x over previous
import functools

import jax
import jax.numpy as jnp
from jax import lax
from jax.experimental import pallas as pl
from jax.experimental.pallas import tpu as pltpu
from jax.experimental.pallas import tpu_sc as plsc

D = 64
K = 50
LANES = 16
CHUNK = 16
CHUNK_IDX = CHUNK * K
IDX_SLICES = [(s * 128, 128) for s in range(CHUNK_IDX // 128)]
if CHUNK_IDX % 128:
    IDX_SLICES.append((CHUNK_IDX - CHUNK_IDX % 128, CHUNK_IDX % 128))


def _make_agg(B: int):
    info = plsc.get_sparse_core_info()
    NC, NS = info.num_cores, info.num_subcores
    NW = NC * NS
    assert B % (NW * CHUNK) == 0
    b_per_w = B // NW
    n_chunks = b_per_w // CHUNK

    mesh = plsc.VectorSubcoreMesh(core_axis_name="c", subcore_axis_name="s")

    @functools.partial(
        pl.kernel,
        mesh=mesh,
        compiler_params=pltpu.CompilerParams(use_tc_tiling_on_sc=False),
        out_type=jax.ShapeDtypeStruct((B, D), jnp.float32),
        scratch_types=[
            pltpu.VMEM((CHUNK_IDX,), jnp.int32),
            pltpu.VMEM((CHUNK_IDX, D), jnp.float32),
            pltpu.VMEM((CHUNK,), jnp.int32),
            pltpu.VMEM((CHUNK, D), jnp.float32),
            pltpu.VMEM((CHUNK, D), jnp.float32),
            pltpu.SemaphoreType.DMA,
        ],
    )
    def agg(item_idx_hbm, nbr_idx_hbm, item_tab_hbm, ent_tab_hbm, out_hbm,
            nidx_v, nrows_v, iidx_v, irows_v, out_v, sem):
        wid = lax.axis_index("s") * NC + lax.axis_index("c")
        row0 = wid * b_per_w

        def chunk_body(g, _):
            r0 = row0 + g * CHUNK
            pltpu.sync_copy(nbr_idx_hbm.at[pl.ds(r0 * K, CHUNK_IDX)], nidx_v)
            pltpu.sync_copy(item_idx_hbm.at[pl.ds(r0, CHUNK)], iidx_v)
            cps = []
            for off, sz in IDX_SLICES:
                cps.append(pltpu.async_copy(
                    ent_tab_hbm.at[nidx_v.at[pl.ds(off, sz)]],
                    nrows_v.at[pl.ds(off, sz)], sem))
            cps.append(pltpu.async_copy(item_tab_hbm.at[iidx_v], irows_v, sem))
            for cp in cps:
                cp.wait()

            def row_body(r, _):
                base = r * K
                for q in range(D // LANES):
                    c = pl.ds(q * LANES, LANES)
                    acc = nrows_v[base, c]
                    for k in range(1, K):
                        acc = acc + nrows_v[base + k, c]
                    out_v[r, c] = irows_v[r, c] + acc * (1.0 / K)
                return 0

            lax.fori_loop(0, CHUNK, row_body, 0)
            pltpu.sync_copy(out_v, out_hbm.at[pl.ds(r0, CHUNK), :])
            return 0

        lax.fori_loop(0, n_chunks, chunk_body, 0)

    return agg


def _fc_body(x_ref, w_ref, b_ref, o_ref):
    y = lax.dot_general(x_ref[...], w_ref[...], (((1,), (1,)), ((), ())),
                        preferred_element_type=jnp.float32)
    o_ref[...] = jnp.maximum(y + b_ref[...], 0.0)


def _make_fc(B: int):
    blk = 2048
    return pl.pallas_call(
        _fc_body,
        grid=(B // blk,),
        in_specs=[
            pl.BlockSpec((blk, D), lambda i: (i, 0)),
            pl.BlockSpec((D, D), lambda i: (0, 0)),
            pl.BlockSpec((1, D), lambda i: (0, 0)),
        ],
        out_specs=pl.BlockSpec((blk, D), lambda i: (i, 0)),
        out_shape=jax.ShapeDtypeStruct((B, D), jnp.float32),
    )


def kernel(item_indices, neighbor_indices, item_table, entity_table, fc1_w, fc1_b):
    B = item_indices.shape[0]
    agg = _make_agg(B)(
        item_indices.astype(jnp.int32),
        neighbor_indices.reshape(-1).astype(jnp.int32),
        item_table, entity_table)
    return _make_fc(B)(agg, fc1_w, fc1_b.reshape(1, D))


# speedup vs baseline: 2.4122x; 1.0960x over previous
import functools

import jax
import jax.numpy as jnp
from jax import lax
from jax.experimental import pallas as pl
from jax.experimental.pallas import tpu as pltpu
from jax.experimental.pallas import tpu_sc as plsc

D = 64
K = 50
LANES = 16
CHUNK = 8
NBUF = 2
CHUNK_IDX = CHUNK * K
IDX_SLICES = [(s * 128, 128) for s in range(CHUNK_IDX // 128)]
if CHUNK_IDX % 128:
    IDX_SLICES.append((CHUNK_IDX - CHUNK_IDX % 128, CHUNK_IDX % 128))


def _make_agg(B: int):
    info = plsc.get_sparse_core_info()
    NC, NS = info.num_cores, info.num_subcores
    NW = NC * NS
    assert B % (NW * CHUNK * NBUF) == 0
    b_per_w = B // NW
    n_chunks = b_per_w // CHUNK

    mesh = plsc.VectorSubcoreMesh(core_axis_name="c", subcore_axis_name="s")

    @functools.partial(
        pl.kernel,
        mesh=mesh,
        compiler_params=pltpu.CompilerParams(use_tc_tiling_on_sc=False),
        out_type=jax.ShapeDtypeStruct((B, D), jnp.float32),
        scratch_types=[
            pltpu.VMEM((b_per_w * K,), jnp.int32),
            pltpu.VMEM((b_per_w,), jnp.int32),
            pltpu.VMEM((NBUF, CHUNK_IDX, D), jnp.float32),
            pltpu.VMEM((NBUF, CHUNK, D), jnp.float32),
            pltpu.VMEM((NBUF, CHUNK, D), jnp.float32),
            [pltpu.SemaphoreType.DMA] * NBUF,
            [pltpu.SemaphoreType.DMA] * NBUF,
        ],
    )
    def agg(item_idx_hbm, nbr_idx_hbm, item_tab_hbm, ent_tab_hbm, out_hbm,
            nidx_v, iidx_v, nrows_v, irows_v, out_v, gsem, osem):
        wid = lax.axis_index("s") * NC + lax.axis_index("c")
        row0 = wid * b_per_w
        pltpu.sync_copy(nbr_idx_hbm.at[pl.ds(row0 * K, b_per_w * K)], nidx_v)
        pltpu.sync_copy(item_idx_hbm.at[pl.ds(row0, b_per_w)], iidx_v)

        def fire(g, b):
            for off, sz in IDX_SLICES:
                pltpu.async_copy(
                    ent_tab_hbm.at[nidx_v.at[pl.ds(g * CHUNK_IDX + off, sz)]],
                    nrows_v.at[b, pl.ds(off, sz)], gsem[b])
            pltpu.async_copy(
                item_tab_hbm.at[iidx_v.at[pl.ds(g * CHUNK, CHUNK)]],
                irows_v.at[b], gsem[b])

        def wait_gather(b):
            pltpu.make_async_copy(
                ent_tab_hbm.at[pl.ds(0, CHUNK_IDX)], nrows_v.at[b], gsem[b]).wait()
            pltpu.make_async_copy(
                item_tab_hbm.at[pl.ds(0, CHUNK)], irows_v.at[b], gsem[b]).wait()

        def wait_out(b):
            pltpu.make_async_copy(
                out_v.at[b], out_hbm.at[pl.ds(0, CHUNK), :], osem[b]).wait()

        for b in range(NBUF):
            fire(b, b)

        @pl.loop(0, n_chunks, step=NBUF)
        def outer(i):
            for b in range(NBUF):
                g = i + b
                wait_gather(b)

                @pl.when(g >= NBUF)
                def _():
                    wait_out(b)

                def row_body(r, _):
                    base = r * K
                    for q in range(D // LANES):
                        c = pl.ds(q * LANES, LANES)
                        acc = nrows_v[b, base, c]
                        for k in range(1, K):
                            acc = acc + nrows_v[b, base + k, c]
                        out_v[b, r, c] = irows_v[b, r, c] + acc * (1.0 / K)
                    return 0

                lax.fori_loop(0, CHUNK, row_body, 0)
                pltpu.async_copy(
                    out_v.at[b], out_hbm.at[pl.ds(row0 + g * CHUNK, CHUNK), :],
                    osem[b])

                @pl.when(g + NBUF < n_chunks)
                def _():
                    fire(g + NBUF, b)

        for b in range(NBUF):
            wait_out(b)

    return agg


def _fc_body(x_ref, w_ref, b_ref, o_ref):
    y = lax.dot_general(x_ref[...], w_ref[...], (((1,), (1,)), ((), ())),
                        preferred_element_type=jnp.float32)
    o_ref[...] = jnp.maximum(y + b_ref[...], 0.0)


def _make_fc(B: int):
    blk = 2048
    return pl.pallas_call(
        _fc_body,
        grid=(B // blk,),
        in_specs=[
            pl.BlockSpec((blk, D), lambda i: (i, 0)),
            pl.BlockSpec((D, D), lambda i: (0, 0)),
            pl.BlockSpec((1, D), lambda i: (0, 0)),
        ],
        out_specs=pl.BlockSpec((blk, D), lambda i: (i, 0)),
        out_shape=jax.ShapeDtypeStruct((B, D), jnp.float32),
    )


def kernel(item_indices, neighbor_indices, item_table, entity_table, fc1_w, fc1_b):
    B = item_indices.shape[0]
    agg = _make_agg(B)(
        item_indices.astype(jnp.int32),
        neighbor_indices.reshape(-1).astype(jnp.int32),
        item_table, entity_table)
    return _make_fc(B)(agg, fc1_w, fc1_b.reshape(1, D))


# speedup vs baseline: 3.0760x; 1.2752x over previous
import functools

import jax
import jax.numpy as jnp
from jax import lax
from jax.experimental import pallas as pl
from jax.experimental.pallas import tpu as pltpu
from jax.experimental.pallas import tpu_sc as plsc

D = 64
K = 50
LANES = 16
CHUNK = 8
NBUF = 2
CHUNK_IDX = CHUNK * K
IDX_SLICES = [(s * 128, 128) for s in range(CHUNK_IDX // 128)]
if CHUNK_IDX % 128:
    IDX_SLICES.append((CHUNK_IDX - CHUNK_IDX % 128, CHUNK_IDX % 128))


def _make_agg(B: int):
    info = plsc.get_sparse_core_info()
    NC, NS = info.num_cores, info.num_subcores
    NW = NC * NS
    assert B % (NW * CHUNK * NBUF) == 0
    b_per_w = B // NW
    n_chunks = b_per_w // CHUNK

    mesh = plsc.VectorSubcoreMesh(core_axis_name="c", subcore_axis_name="s")

    @functools.partial(
        pl.kernel,
        mesh=mesh,
        compiler_params=pltpu.CompilerParams(use_tc_tiling_on_sc=False),
        out_type=jax.ShapeDtypeStruct((B, D), jnp.float32),
        scratch_types=[
            pltpu.VMEM((b_per_w * K,), jnp.int32),
            pltpu.VMEM((b_per_w,), jnp.int32),
            pltpu.VMEM((NBUF, CHUNK_IDX, D), jnp.float32),
            pltpu.VMEM((NBUF, CHUNK, D), jnp.float32),
            pltpu.VMEM((NBUF, CHUNK, D), jnp.float32),
            [pltpu.SemaphoreType.DMA] * NBUF,
            [pltpu.SemaphoreType.DMA] * NBUF,
        ],
    )
    def agg(item_idx_hbm, nbr_idx_hbm, item_tab_hbm, ent_tab_hbm, out_hbm,
            nidx_v, iidx_v, nrows_v, irows_v, out_v, gsem, osem):
        wid = lax.axis_index("s") * NC + lax.axis_index("c")
        row0 = wid * b_per_w
        pltpu.sync_copy(nbr_idx_hbm.at[pl.ds(row0 * K, b_per_w * K)], nidx_v)
        pltpu.sync_copy(item_idx_hbm.at[pl.ds(row0, b_per_w)], iidx_v)

        def fire(g, b):
            for off, sz in IDX_SLICES:
                pltpu.async_copy(
                    ent_tab_hbm.at[nidx_v.at[pl.ds(g * CHUNK_IDX + off, sz)]],
                    nrows_v.at[b, pl.ds(off, sz)], gsem[b])
            pltpu.async_copy(
                item_tab_hbm.at[iidx_v.at[pl.ds(g * CHUNK, CHUNK)]],
                irows_v.at[b], gsem[b])

        def wait_gather(b):
            pltpu.make_async_copy(
                ent_tab_hbm.at[pl.ds(0, CHUNK_IDX)], nrows_v.at[b], gsem[b]).wait()
            pltpu.make_async_copy(
                item_tab_hbm.at[pl.ds(0, CHUNK)], irows_v.at[b], gsem[b]).wait()

        def wait_out(b):
            pltpu.make_async_copy(
                out_v.at[b], out_hbm.at[pl.ds(0, CHUNK), :], osem[b]).wait()

        for b in range(NBUF):
            fire(b, b)

        @pl.loop(0, n_chunks, step=NBUF)
        def outer(i):
            for b in range(NBUF):
                g = i + b
                wait_gather(b)

                @pl.when(g >= NBUF)
                def _():
                    wait_out(b)

                def row_body(r, _):
                    base = r * K
                    for q in range(D // LANES):
                        c = pl.ds(q * LANES, LANES)
                        acc = nrows_v[b, base, c]
                        for k in range(1, K):
                            acc = acc + nrows_v[b, base + k, c]
                        out_v[b, r, c] = irows_v[b, r, c] + acc * (1.0 / K)
                    return 0

                lax.fori_loop(0, CHUNK, row_body, 0)
                pltpu.async_copy(
                    out_v.at[b], out_hbm.at[pl.ds(row0 + g * CHUNK, CHUNK), :],
                    osem[b])

                @pl.when(g + NBUF < n_chunks)
                def _():
                    fire(g + NBUF, b)

        for b in range(NBUF):
            wait_out(b)

    return agg


TBLK = 4096
TBLK_BITS = 12


def _tr_body(x_ref, o_ref):
    t = x_ref[...].T
    h = pl.program_id(1)

    @pl.when(h == 0)
    def _():
        o_ref[:, 0:D] = t

    @pl.when(h == 1)
    def _():
        o_ref[:, D:2 * D] = t


def _make_tr(R: int):
    npairs = (R + 2 * TBLK - 1) // (2 * TBLK)
    nblocks = (R + TBLK - 1) // TBLK
    return pl.pallas_call(
        _tr_body,
        grid=(npairs, 2),
        in_specs=[pl.BlockSpec(
            (D, TBLK),
            lambda j, h: (0, jnp.minimum(2 * j + h, nblocks - 1)))],
        out_specs=pl.BlockSpec((TBLK, 2 * D), lambda j, h: (j, 0)),
        out_shape=jax.ShapeDtypeStruct((npairs * TBLK, 2 * D), jnp.float32),
    )


def _remap(i):
    blk = i >> TBLK_BITS
    return ((((blk >> 1) << TBLK_BITS) + (i & (TBLK - 1))) << 1) + (blk & 1)


def _fc_body(x_ref, w_ref, b_ref, o_ref):
    y = lax.dot_general(x_ref[...], w_ref[...], (((1,), (1,)), ((), ())),
                        preferred_element_type=jnp.float32)
    o_ref[...] = jnp.maximum(y + b_ref[...], 0.0)


def _make_fc(B: int):
    blk = 2048
    return pl.pallas_call(
        _fc_body,
        grid=(B // blk,),
        in_specs=[
            pl.BlockSpec((blk, D), lambda i: (i, 0)),
            pl.BlockSpec((D, D), lambda i: (0, 0)),
            pl.BlockSpec((1, D), lambda i: (0, 0)),
        ],
        out_specs=pl.BlockSpec((blk, D), lambda i: (i, 0)),
        out_shape=jax.ShapeDtypeStruct((B, D), jnp.float32),
    )


def kernel(item_indices, neighbor_indices, item_table, entity_table, fc1_w, fc1_b):
    B = item_indices.shape[0]
    NE = entity_table.shape[0]
    NI = item_table.shape[0]
    ent_lin = _make_tr(NE)(entity_table.T)
    ent_lin = ent_lin.reshape(2 * ent_lin.shape[0], D)
    itm_lin = _make_tr(NI)(item_table.T)
    itm_lin = itm_lin.reshape(2 * itm_lin.shape[0], D)
    nb = _remap(neighbor_indices.reshape(-1).astype(jnp.int32))
    ii = _remap(item_indices.astype(jnp.int32))
    agg = _make_agg(B)(ii, nb, itm_lin, ent_lin)
    return _make_fc(B)(agg, fc1_w, fc1_b.reshape(1, D))


# speedup vs baseline: 4.2985x; 1.3974x over previous
import functools

import jax
import jax.numpy as jnp
from jax import lax
from jax.experimental import pallas as pl
from jax.experimental.pallas import tpu as pltpu
from jax.experimental.pallas import tpu_sc as plsc

D = 64
K = 50
LANES = 16
CHUNK = 8
NBUF = 2
CHUNK_IDX = CHUNK * K
IDX_SLICES = [(s * 128, 128) for s in range(CHUNK_IDX // 128)]
if CHUNK_IDX % 128:
    IDX_SLICES.append((CHUNK_IDX - CHUNK_IDX % 128, CHUNK_IDX % 128))


def _make_agg(B: int):
    info = plsc.get_sparse_core_info()
    NC, NS = info.num_cores, info.num_subcores
    NW = NC * NS
    assert B % (NW * CHUNK * NBUF) == 0
    b_per_w = B // NW
    n_chunks = b_per_w // CHUNK

    mesh = plsc.VectorSubcoreMesh(core_axis_name="c", subcore_axis_name="s")

    @functools.partial(
        pl.kernel,
        mesh=mesh,
        compiler_params=pltpu.CompilerParams(use_tc_tiling_on_sc=False),
        out_type=jax.ShapeDtypeStruct((B, D), jnp.float32),
        scratch_types=[
            pltpu.VMEM((b_per_w * K,), jnp.int32),
            pltpu.VMEM((b_per_w,), jnp.int32),
            pltpu.VMEM((NBUF, CHUNK_IDX, D), jnp.float32),
            pltpu.VMEM((NBUF, CHUNK, D), jnp.float32),
            pltpu.VMEM((NBUF, CHUNK, D), jnp.float32),
            [pltpu.SemaphoreType.DMA] * NBUF,
            [pltpu.SemaphoreType.DMA] * NBUF,
        ],
    )
    def agg(item_idx_hbm, nbr_idx_hbm, item_tab_hbm, ent_tab_hbm, out_hbm,
            nidx_v, iidx_v, nrows_v, irows_v, out_v, gsem, osem):
        wid = lax.axis_index("s") * NC + lax.axis_index("c")
        row0 = wid * b_per_w
        pltpu.sync_copy(nbr_idx_hbm.at[pl.ds(row0 * K, b_per_w * K)], nidx_v)
        pltpu.sync_copy(item_idx_hbm.at[pl.ds(row0, b_per_w)], iidx_v)

        def fire(g, b):
            for off, sz in IDX_SLICES:
                pltpu.async_copy(
                    ent_tab_hbm.at[nidx_v.at[pl.ds(g * CHUNK_IDX + off, sz)]],
                    nrows_v.at[b, pl.ds(off, sz)], gsem[b])
            pltpu.async_copy(
                item_tab_hbm.at[iidx_v.at[pl.ds(g * CHUNK, CHUNK)]],
                irows_v.at[b], gsem[b])

        def wait_gather(b):
            pltpu.make_async_copy(
                ent_tab_hbm.at[pl.ds(0, CHUNK_IDX)], nrows_v.at[b], gsem[b]).wait()
            pltpu.make_async_copy(
                item_tab_hbm.at[pl.ds(0, CHUNK)], irows_v.at[b], gsem[b]).wait()

        def wait_out(b):
            pltpu.make_async_copy(
                out_v.at[b], out_hbm.at[pl.ds(0, CHUNK), :], osem[b]).wait()

        for b in range(NBUF):
            fire(b, b)

        @pl.loop(0, n_chunks, step=NBUF)
        def outer(i):
            for b in range(NBUF):
                g = i + b
                wait_gather(b)

                @pl.when(g >= NBUF)
                def _():
                    wait_out(b)

                def row_body(r, _):
                    base = r * K
                    for q in range(D // LANES):
                        c = pl.ds(q * LANES, LANES)
                        acc = nrows_v[b, base, c]
                        for k in range(1, K):
                            acc = acc + nrows_v[b, base + k, c]
                        out_v[b, r, c] = irows_v[b, r, c] + acc * (1.0 / K)
                    return 0

                lax.fori_loop(0, CHUNK, row_body, 0)
                pltpu.async_copy(
                    out_v.at[b], out_hbm.at[pl.ds(row0 + g * CHUNK, CHUNK), :],
                    osem[b])

                @pl.when(g + NBUF < n_chunks)
                def _():
                    fire(g + NBUF, b)

        for b in range(NBUF):
            wait_out(b)

    return agg


TBLK = 8192
TBLK_BITS = 13


def _tr_body(a_ref, b_ref, o_ref):
    o_ref[...] = jnp.concatenate([a_ref[...].T, b_ref[...].T], axis=1)


def _make_tr(R: int):
    npairs = (R + 2 * TBLK - 1) // (2 * TBLK)
    nblocks = (R + TBLK - 1) // TBLK
    return pl.pallas_call(
        _tr_body,
        grid=(npairs,),
        in_specs=[
            pl.BlockSpec((D, TBLK), lambda j: (0, 2 * j)),
            pl.BlockSpec((D, TBLK),
                         lambda j: (0, jnp.minimum(2 * j + 1, nblocks - 1))),
        ],
        out_specs=pl.BlockSpec((TBLK, 2 * D), lambda j: (j, 0)),
        out_shape=jax.ShapeDtypeStruct((npairs * TBLK, 2 * D), jnp.float32),
    )


def _remap(i):
    blk = i >> TBLK_BITS
    return ((((blk >> 1) << TBLK_BITS) + (i & (TBLK - 1))) << 1) + (blk & 1)


def _fc_body(x_ref, w_ref, b_ref, o_ref):
    y = lax.dot_general(x_ref[...], w_ref[...], (((1,), (1,)), ((), ())),
                        preferred_element_type=jnp.float32)
    o_ref[...] = jnp.maximum(y + b_ref[...], 0.0)


def _make_fc(B: int):
    blk = 2048
    return pl.pallas_call(
        _fc_body,
        grid=(B // blk,),
        in_specs=[
            pl.BlockSpec((blk, D), lambda i: (i, 0)),
            pl.BlockSpec((D, D), lambda i: (0, 0)),
            pl.BlockSpec((1, D), lambda i: (0, 0)),
        ],
        out_specs=pl.BlockSpec((blk, D), lambda i: (i, 0)),
        out_shape=jax.ShapeDtypeStruct((B, D), jnp.float32),
    )


def kernel(item_indices, neighbor_indices, item_table, entity_table, fc1_w, fc1_b):
    B = item_indices.shape[0]
    NE = entity_table.shape[0]
    NI = item_table.shape[0]
    ent_t = entity_table.T
    itm_t = item_table.T
    ent_lin = _make_tr(NE)(ent_t, ent_t)
    ent_lin = ent_lin.reshape(2 * ent_lin.shape[0], D)
    itm_lin = _make_tr(NI)(itm_t, itm_t)
    itm_lin = itm_lin.reshape(2 * itm_lin.shape[0], D)
    nb = _remap(neighbor_indices.reshape(-1).astype(jnp.int32))
    ii = _remap(item_indices.astype(jnp.int32))
    agg = _make_agg(B)(ii, nb, itm_lin, ent_lin)
    return _make_fc(B)(agg, fc1_w, fc1_b.reshape(1, D))
